# baseline (device time: 217131 ns/iter reference)
import jax
import jax.numpy as jnp
from jax import lax
from jax.experimental import pallas as pl
from jax.experimental.pallas import tpu as pltpu

N_DEV = 16
M = 4096
K_SHARD = 256
N_OUT = 2048
CHUNK = M // N_DEV
N_LANES = 8
LANE_COLS = N_OUT // N_LANES
NSLOT = 4
COMM_DTYPE = jnp.bfloat16

MESH = pl.DeviceIdType.MESH
N_STEPS = 2 * (N_DEV - 1)

RING = (0, 1, 5, 9, 13, 14, 10, 6, 2, 3, 7, 11, 15, 12, 8, 4)

STREAMS = (
    (+1, 0 * LANE_COLS), (-1, 1 * LANE_COLS),
    (+1, 2 * LANE_COLS), (-1, 3 * LANE_COLS),
    (+1, 4 * LANE_COLS), (-1, 5 * LANE_COLS),
    (+1, 6 * LANE_COLS), (-1, 7 * LANE_COLS),
)


def _body(x_ref, w_ref, sx_ref, sw_ref, meta_ref, out_ref, p_ref, w16_ref,
          b0, b1, b2, b3, b4, b5, b6, b7,
          ss0, rs0, ss1, rs1, ss2, rs2, ss3, rs3,
          ss4, rs4, ss5, rs5, ss6, rs6, ss7, rs7,
          cr_cw, cr_ccw):
    pos = meta_ref[0]
    left = meta_ref[1]
    right = meta_ref[2]
    bufs = (b0, b1, b2, b3, b4, b5, b6, b7)
    send_sems = (ss0, ss1, ss2, ss3, ss4, ss5, ss6, ss7)
    recv_sems = (rs0, rs1, rs2, rs3, rs4, rs5, rs6, rs7)

    def credit_of(k):
        return cr_cw if STREAMS[k][0] > 0 else cr_ccw

    barrier = pltpu.get_barrier_semaphore()
    for nbr in (left, right):
        pl.semaphore_signal(barrier, inc=1, device_id=(nbr,),
                            device_id_type=MESH)
    pl.semaphore_wait(barrier, 2)

    scale = sx_ref[0] * sw_ref[0]
    w16_ref[:, :] = w_ref[:, :].astype(jnp.bfloat16)

    def gemm_chunk(c):
        return lax.dot_general(
            x_ref[pl.ds(c * CHUNK, CHUNK), :].astype(jnp.bfloat16),
            w16_ref[:, :],
            (((1,), (0,)), ((), ())),
            preferred_element_type=jnp.float32,
        ).astype(COMM_DTYPE)

    def rows(c, k):
        return (pl.ds(c * CHUNK, CHUNK), pl.ds(STREAMS[k][1], LANE_COLS))

    def chunk_of(k, g):
        d, _ = STREAMS[k]
        if g < N_DEV - 1:
            return (pos - d * (g + 1)) % N_DEV
        t = g - (N_DEV - 1)
        return (pos - d * t) % N_DEV

    def mk(k, g):
        d, _ = STREAMS[k]
        return pltpu.make_async_remote_copy(
            src_ref=bufs[k].at[g % NSLOT],
            dst_ref=bufs[k].at[(g + 1) % NSLOT],
            send_sem=send_sems[k].at[g % NSLOT],
            recv_sem=recv_sems[k].at[(g + 1) % NSLOT],
            device_id=(right,) if d > 0 else (left,),
            device_id_type=MESH,
        )

    seed = gemm_chunk(pos)
    for k in range(N_LANES):
        c0 = STREAMS[k][1]
        bufs[k][0, :, :] = seed[:, c0:c0 + LANE_COLS]
        mk(k, 0).start()

    for g in range(N_STEPS):
        slot_r = (g + 1) % NSLOT
        rdmas = [mk(k, g) for k in range(N_LANES)]
        is_rs = g < N_DEV - 1
        finalize = g == N_DEV - 2
        if g <= 7:
            p_ref[(pl.ds(((pos - g - 1) % N_DEV) * CHUNK, CHUNK),
                   slice(None))] = gemm_chunk((pos - g - 1) % N_DEV)
        if g <= 6:
            p_ref[(pl.ds(((pos + g + 1) % N_DEV) * CHUNK, CHUNK),
                   slice(None))] = gemm_chunk((pos + g + 1) % N_DEV)
        accs = [None] * N_LANES
        credited = set()
        for k in range(N_LANES):
            rdmas[k].wait_recv()
            if is_rs:
                acc = (bufs[k][slot_r, :, :]
                       + p_ref[rows(chunk_of(k, g), k)])
                bufs[k][slot_r, :, :] = acc
                accs[k] = acc
            if g + 1 < N_STEPS:
                d = STREAMS[k][0]
                if g + 1 >= NSLOT - 1 and d not in credited:
                    pl.semaphore_wait(credit_of(k), 1)
                    credited.add(d)
                mk(k, g + 1).start()
        for k in range(N_LANES):
            rdmas[k].wait_send()
        if g <= N_STEPS - NSLOT:
            pl.semaphore_signal(cr_cw, inc=1, device_id=(left,),
                                device_id_type=MESH)
            pl.semaphore_signal(cr_ccw, inc=1, device_id=(right,),
                                device_id_type=MESH)
        if finalize:
            for k in range(N_LANES):
                out_ref[rows(chunk_of(k, g), k)] = (
                    accs[k].astype(jnp.float32) * scale)
        if not is_rs:
            for k in range(N_LANES):
                out_ref[rows(chunk_of(k, g), k)] = (
                    bufs[k][slot_r, :, :].astype(jnp.float32) * scale)


def kernel(x, w_mat, scale_x, scale_w):
    ring = jnp.array(RING, dtype=jnp.int32)
    my = lax.axis_index("i")
    pos = jnp.argmax(ring == my).astype(jnp.int32)
    meta = jnp.stack([pos, ring[(pos - 1) % N_DEV], ring[(pos + 1) % N_DEV]])
    comm = pltpu.VMEM((NSLOT, CHUNK, LANE_COLS), COMM_DTYPE)
    dma = pltpu.SemaphoreType.DMA((NSLOT,))
    return pl.pallas_call(
        _body,
        out_shape=jax.ShapeDtypeStruct((M, N_OUT), jnp.float32),
        in_specs=[
            pl.BlockSpec(memory_space=pltpu.VMEM),
            pl.BlockSpec(memory_space=pltpu.VMEM),
            pl.BlockSpec(memory_space=pltpu.SMEM),
            pl.BlockSpec(memory_space=pltpu.SMEM),
            pl.BlockSpec(memory_space=pltpu.SMEM),
        ],
        out_specs=pl.BlockSpec(memory_space=pltpu.VMEM),
        scratch_shapes=[
            pltpu.VMEM((M, N_OUT), COMM_DTYPE),
            pltpu.VMEM((K_SHARD, N_OUT), COMM_DTYPE),
            comm, comm, comm, comm, comm, comm, comm, comm,
            dma, dma, dma, dma, dma, dma, dma, dma,
            dma, dma, dma, dma, dma, dma, dma, dma,
            pltpu.SemaphoreType.REGULAR,
            pltpu.SemaphoreType.REGULAR,
        ],
        compiler_params=pltpu.CompilerParams(
            collective_id=0,
            vmem_limit_bytes=100 * 1024 * 1024,
        ),
    )(x, w_mat, scale_x, scale_w, meta)


# device time: 216891 ns/iter; 1.0011x vs baseline; 1.0011x over previous
import jax
import jax.numpy as jnp
from jax import lax
from jax.experimental import pallas as pl
from jax.experimental.pallas import tpu as pltpu

N_DEV = 16
M = 4096
K_SHARD = 256
N_OUT = 2048
CHUNK = M // N_DEV
N_LANES = 8
LANE_COLS = N_OUT // N_LANES
NSLOT = 3
COMM_DTYPE = jnp.bfloat16

MESH = pl.DeviceIdType.MESH
N_STEPS = 2 * (N_DEV - 1)

RING = (0, 1, 5, 9, 13, 14, 10, 6, 2, 3, 7, 11, 15, 12, 8, 4)

STREAMS = (
    (+1, 0 * LANE_COLS), (-1, 1 * LANE_COLS),
    (+1, 2 * LANE_COLS), (-1, 3 * LANE_COLS),
    (+1, 4 * LANE_COLS), (-1, 5 * LANE_COLS),
    (+1, 6 * LANE_COLS), (-1, 7 * LANE_COLS),
)


def _body(x_ref, w_ref, sx_ref, sw_ref, meta_ref, out_ref, p_ref, w16_ref,
          b0, b1, b2, b3, b4, b5, b6, b7,
          ss0, rs0, ss1, rs1, ss2, rs2, ss3, rs3,
          ss4, rs4, ss5, rs5, ss6, rs6, ss7, rs7,
          cr_cw, cr_ccw):
    pos = meta_ref[0]
    left = meta_ref[1]
    right = meta_ref[2]
    bufs = (b0, b1, b2, b3, b4, b5, b6, b7)
    send_sems = (ss0, ss1, ss2, ss3, ss4, ss5, ss6, ss7)
    recv_sems = (rs0, rs1, rs2, rs3, rs4, rs5, rs6, rs7)

    def credit_of(k):
        return cr_cw if STREAMS[k][0] > 0 else cr_ccw

    barrier = pltpu.get_barrier_semaphore()
    for nbr in (left, right):
        pl.semaphore_signal(barrier, inc=1, device_id=(nbr,),
                            device_id_type=MESH)
    pl.semaphore_wait(barrier, 2)

    scale = sx_ref[0] * sw_ref[0]
    w16_ref[:, :] = w_ref[:, :].astype(jnp.bfloat16)

    def gemm_chunk(c):
        return lax.dot_general(
            x_ref[pl.ds(c * CHUNK, CHUNK), :].astype(jnp.bfloat16),
            w16_ref[:, :],
            (((1,), (0,)), ((), ())),
            preferred_element_type=jnp.float32,
        ).astype(COMM_DTYPE)

    def rows(c, k):
        return (pl.ds(c * CHUNK, CHUNK), pl.ds(STREAMS[k][1], LANE_COLS))

    def chunk_of(k, g):
        d, _ = STREAMS[k]
        if g < N_DEV - 1:
            return (pos - d * (g + 1)) % N_DEV
        t = g - (N_DEV - 1)
        return (pos - d * t) % N_DEV

    def mk(k, g):
        d, _ = STREAMS[k]
        return pltpu.make_async_remote_copy(
            src_ref=bufs[k].at[g % NSLOT],
            dst_ref=bufs[k].at[(g + 1) % NSLOT],
            send_sem=send_sems[k].at[g % NSLOT],
            recv_sem=recv_sems[k].at[(g + 1) % NSLOT],
            device_id=(right,) if d > 0 else (left,),
            device_id_type=MESH,
        )

    seed = gemm_chunk(pos)
    for k in range(N_LANES):
        c0 = STREAMS[k][1]
        bufs[k][0, :, :] = seed[:, c0:c0 + LANE_COLS]
        mk(k, 0).start()

    for g in range(N_STEPS):
        slot_r = (g + 1) % NSLOT
        rdmas = [mk(k, g) for k in range(N_LANES)]
        is_rs = g < N_DEV - 1
        finalize = g == N_DEV - 2
        if g <= 7:
            p_ref[(pl.ds(((pos - g - 1) % N_DEV) * CHUNK, CHUNK),
                   slice(None))] = gemm_chunk((pos - g - 1) % N_DEV)
        if g <= 6:
            p_ref[(pl.ds(((pos + g + 1) % N_DEV) * CHUNK, CHUNK),
                   slice(None))] = gemm_chunk((pos + g + 1) % N_DEV)
        accs = [None] * N_LANES
        credited = set()
        for k in range(N_LANES):
            rdmas[k].wait_recv()
            if is_rs:
                acc = (bufs[k][slot_r, :, :]
                       + p_ref[rows(chunk_of(k, g), k)])
                bufs[k][slot_r, :, :] = acc
                accs[k] = acc
            if g + 1 < N_STEPS:
                d = STREAMS[k][0]
                if g + 1 >= NSLOT - 1 and d not in credited:
                    pl.semaphore_wait(credit_of(k), 1)
                    credited.add(d)
                mk(k, g + 1).start()
        for k in range(N_LANES):
            rdmas[k].wait_send()
        if g <= N_STEPS - NSLOT:
            pl.semaphore_signal(cr_cw, inc=1, device_id=(left,),
                                device_id_type=MESH)
            pl.semaphore_signal(cr_ccw, inc=1, device_id=(right,),
                                device_id_type=MESH)
        if finalize:
            for k in range(N_LANES):
                out_ref[rows(chunk_of(k, g), k)] = (
                    accs[k].astype(jnp.float32) * scale)
        if not is_rs:
            for k in range(N_LANES):
                out_ref[rows(chunk_of(k, g), k)] = (
                    bufs[k][slot_r, :, :].astype(jnp.float32) * scale)


def kernel(x, w_mat, scale_x, scale_w):
    ring = jnp.array(RING, dtype=jnp.int32)
    my = lax.axis_index("i")
    pos = jnp.argmax(ring == my).astype(jnp.int32)
    meta = jnp.stack([pos, ring[(pos - 1) % N_DEV], ring[(pos + 1) % N_DEV]])
    comm = pltpu.VMEM((NSLOT, CHUNK, LANE_COLS), COMM_DTYPE)
    dma = pltpu.SemaphoreType.DMA((NSLOT,))
    return pl.pallas_call(
        _body,
        out_shape=jax.ShapeDtypeStruct((M, N_OUT), jnp.float32),
        in_specs=[
            pl.BlockSpec(memory_space=pltpu.VMEM),
            pl.BlockSpec(memory_space=pltpu.VMEM),
            pl.BlockSpec(memory_space=pltpu.SMEM),
            pl.BlockSpec(memory_space=pltpu.SMEM),
            pl.BlockSpec(memory_space=pltpu.SMEM),
        ],
        out_specs=pl.BlockSpec(memory_space=pltpu.VMEM),
        scratch_shapes=[
            pltpu.VMEM((M, N_OUT), COMM_DTYPE),
            pltpu.VMEM((K_SHARD, N_OUT), COMM_DTYPE),
            comm, comm, comm, comm, comm, comm, comm, comm,
            dma, dma, dma, dma, dma, dma, dma, dma,
            dma, dma, dma, dma, dma, dma, dma, dma,
            pltpu.SemaphoreType.REGULAR,
            pltpu.SemaphoreType.REGULAR,
        ],
        compiler_params=pltpu.CompilerParams(
            collective_id=0,
            vmem_limit_bytes=100 * 1024 * 1024,
        ),
    )(x, w_mat, scale_x, scale_w, meta)
